# Initial kernel scaffold; baseline (speedup 1.0000x reference)
import jax
import jax.numpy as jnp
from jax.experimental import pallas as pl


def kernel(gene_feat, reg_w, gp_w, patches, Wr, br, Wp, bp, ln1_g, ln1_b, Wo, bo, Wl, bl, Wm, bm, ln2_g, ln2_b, Wf1, bf1, Wf2, bf2, reg_src, reg_dst, gp_src, gp_dst):
    raise NotImplementedError("write your pallas kernel here")


# speedup vs baseline: 2.6642x; 2.6642x over previous
import functools

import jax
import jax.numpy as jnp
from jax import lax
from jax.experimental import pallas as pl
from jax.experimental.pallas import tpu as pltpu
from jax.experimental.pallas import tpu_sc as plsc

HID = 128
NBINS = 4
NGENE = 10000
NPAT = 64
EREG = 320000
EGP = 160000
NPATCH = 512
PDIM = 768

NC = 2
NS = 16
LANES = 16


def _gelu(x):
    return 0.5 * x * (1.0 + lax.erf(x * 0.7071067811865476))


def _make_seg_sum(n_idx_rows, nseg, nfeat_rows, CH):
    rows_per_tile = n_idx_rows // (NC * NS)
    assert rows_per_tile * NC * NS == n_idx_rows
    n_chunks = rows_per_tile // CH
    assert n_chunks * CH == rows_per_tile
    zrows = nseg // NS
    assert zrows * NS == nseg
    CROWS = CH * 128

    mesh = plsc.VectorSubcoreMesh(core_axis_name="c", subcore_axis_name="s",
                                  num_cores=NC, num_subcores=NS)

    @functools.partial(
        pl.kernel,
        out_type=jax.ShapeDtypeStruct((NC, nseg, HID), jnp.float32),
        mesh=mesh,
        scratch_types=[
            pltpu.VMEM((CH, 128), jnp.int32),
            pltpu.VMEM((CH, 128), jnp.int32),
            pltpu.VMEM((CROWS,), jnp.float32),
            pltpu.VMEM((CROWS, HID), jnp.float32),
            pltpu.VMEM_SHARED((nseg, HID), jnp.float32),
            pltpu.SemaphoreType.DMA,
        ],
    )
    def seg_sum(feat_hbm, src_hbm, dst_hbm, w_hbm, out_hbm,
                src_v, dst_v, w_v, rows_v, acc_sh, sem):
        c = lax.axis_index("c")
        s = lax.axis_index("s")
        tile = c * NS + s
        row_base = tile * rows_per_tile

        def zbody(r, carry):
            for j in range(HID // LANES):
                rows_v[r, pl.ds(j * LANES, LANES)] = jnp.zeros((LANES,),
                                                               jnp.float32)
            return carry
        lax.fori_loop(0, CROWS, zbody, 0)
        off = 0
        rem = zrows
        while rem > 0:
            n = min(rem, CROWS)
            pltpu.sync_copy(rows_v.at[pl.ds(0, n)],
                            acc_sh.at[pl.ds(s * zrows + off, n)])
            off += n
            rem -= n
        plsc.subcore_barrier()

        def chunk(i, carry):
            rb = row_base + i * CH
            eb = rb * 128
            pltpu.sync_copy(src_hbm.at[pl.ds(rb, CH)], src_v)
            pltpu.sync_copy(dst_hbm.at[pl.ds(rb, CH)], dst_v)
            pltpu.sync_copy(w_hbm.at[pl.ds(eb, CROWS)], w_v)
            descs = [
                pltpu.async_copy(feat_hbm.at[src_v.at[j]],
                                 rows_v.at[pl.ds(j * 128, 128)], sem)
                for j in range(CH)
            ]
            for d in descs:
                d.wait()

            def sbody(g, carry2):
                r0 = g * LANES
                w16 = w_v[pl.ds(r0, LANES)]
                for k in range(LANES):
                    wk = w16[k]
                    for j in range(HID // LANES):
                        sl = pl.ds(j * LANES, LANES)
                        rows_v[r0 + k, sl] = rows_v[r0 + k, sl] * wk
                return carry2
            lax.fori_loop(0, CROWS // LANES, sbody, 0)

            for j in range(CH):
                pltpu.sync_copy(rows_v.at[pl.ds(j * 128, 128)],
                                acc_sh.at[dst_v.at[j]], add=True)
            return carry
        lax.fori_loop(0, n_chunks, chunk, 0)

        plsc.subcore_barrier()
        off = 0
        rem = zrows
        while rem > 0:
            n = min(rem, CROWS)
            pltpu.sync_copy(acc_sh.at[pl.ds(s * zrows + off, n)],
                            out_hbm.at[c, pl.ds(s * zrows + off, n)])
            off += n
            rem -= n

    return seg_sum


_seg_sum_reg = _make_seg_sum(n_idx_rows=2560, nseg=10240, nfeat_rows=NGENE,
                             CH=2)
_seg_sum_gp = _make_seg_sum(n_idx_rows=1280, nseg=NPAT, nfeat_rows=NGENE,
                            CH=4)


def _hgene_body(agg_ref, gene_ref, wr_ref, br_ref, out_ref):
    a = agg_ref[0] + agg_ref[1]
    t = jnp.dot(a, wr_ref[...], preferred_element_type=jnp.float32)
    out_ref[...] = gene_ref[...] + _gelu(t + br_ref[...])


def _hgene(agg2, gene, Wr, br2):
    BR = 1000
    return pl.pallas_call(
        _hgene_body,
        grid=(NGENE // BR,),
        in_specs=[
            pl.BlockSpec((NC, BR, HID), lambda i: (0, i, 0)),
            pl.BlockSpec((BR, HID), lambda i: (i, 0)),
            pl.BlockSpec((HID, HID), lambda i: (0, 0)),
            pl.BlockSpec((1, HID), lambda i: (0, 0)),
        ],
        out_specs=pl.BlockSpec((BR, HID), lambda i: (i, 0)),
        out_shape=jax.ShapeDtypeStruct((NGENE, HID), jnp.float32),
    )(agg2, gene, Wr, br2)


def _wsi_body(p_ref, wl_ref, bl_ref, wm_ref, bm_ref, g_ref, b_ref, out_ref):
    x = p_ref[0]
    m = jnp.sum(x, axis=0, keepdims=True) * (1.0 / NPATCH)
    v = _gelu(jnp.dot(m, wl_ref[...], preferred_element_type=jnp.float32)
              + bl_ref[...])
    v = _gelu(jnp.dot(v, wm_ref[...], preferred_element_type=jnp.float32)
              + bm_ref[...])
    mu = jnp.mean(v, axis=-1, keepdims=True)
    var = jnp.mean((v - mu) ** 2, axis=-1, keepdims=True)
    out_ref[0] = (v - mu) / jnp.sqrt(var + 1e-5) * g_ref[...] + b_ref[...]


def _wsi(patches, Wl, bl2, Wm, bm2, g2, b2):
    return pl.pallas_call(
        _wsi_body,
        grid=(NPAT,),
        in_specs=[
            pl.BlockSpec((1, NPATCH, PDIM), lambda i: (i, 0, 0)),
            pl.BlockSpec((PDIM, HID), lambda i: (0, 0)),
            pl.BlockSpec((1, HID), lambda i: (0, 0)),
            pl.BlockSpec((HID, HID), lambda i: (0, 0)),
            pl.BlockSpec((1, HID), lambda i: (0, 0)),
            pl.BlockSpec((1, HID), lambda i: (0, 0)),
            pl.BlockSpec((1, HID), lambda i: (0, 0)),
        ],
        out_specs=pl.BlockSpec((1, 1, HID), lambda i: (i, 0, 0)),
        out_shape=jax.ShapeDtypeStruct((NPAT, 1, HID), jnp.float32),
    )(patches, Wl, bl2, Wm, bm2, g2, b2)


def _fusion_body(acc_ref, wsi_ref, wp_ref, bp_ref, g1_ref, b1_ref,
                 wo_ref, bo_ref, wf1a_ref, wf1b_ref, bf1_ref,
                 wf2_ref, bf2_ref, logits_ref, omics_ref):
    acc = acc_ref[0] + acc_ref[1]
    v = _gelu(jnp.dot(acc, wp_ref[...], preferred_element_type=jnp.float32)
              + bp_ref[...])
    mu = jnp.mean(v, axis=-1, keepdims=True)
    var = jnp.mean((v - mu) ** 2, axis=-1, keepdims=True)
    pat = (v - mu) / jnp.sqrt(var + 1e-5) * g1_ref[...] + b1_ref[...]
    omics_ref[...] = (jnp.dot(pat, wo_ref[...],
                              preferred_element_type=jnp.float32)
                      + bo_ref[...])
    h = _gelu(jnp.dot(pat, wf1a_ref[...], preferred_element_type=jnp.float32)
              + jnp.dot(wsi_ref[...], wf1b_ref[...],
                        preferred_element_type=jnp.float32)
              + bf1_ref[...])
    logits_ref[...] = (jnp.dot(h, wf2_ref[...],
                               preferred_element_type=jnp.float32)
                       + bf2_ref[...])


def _fusion(accp, wsi, Wp, bp2, g12, b12, Wo, bo2, Wf1a, Wf1b, bf12,
            Wf2, bf22):
    return pl.pallas_call(
        _fusion_body,
        out_shape=(jax.ShapeDtypeStruct((NPAT, NBINS), jnp.float32),
                   jax.ShapeDtypeStruct((NPAT, NBINS), jnp.float32)),
    )(accp, wsi, Wp, bp2, g12, b12, Wo, bo2, Wf1a, Wf1b, bf12, Wf2, bf22)


def _pad_idx(x, n_rows):
    pad = n_rows * 128 - x.shape[0]
    x = jnp.concatenate([x.astype(jnp.int32), jnp.zeros((pad,), jnp.int32)])
    return x.reshape(n_rows, 128)


def _pad_w(w, n_rows):
    pad = n_rows * 128 - w.shape[0]
    return jnp.concatenate([w, jnp.zeros((pad,), jnp.float32)])


def kernel(gene_feat, reg_w, gp_w, patches, Wr, br, Wp, bp, ln1_g, ln1_b,
           Wo, bo, Wl, bl, Wm, bm, ln2_g, ln2_b, Wf1, bf1, Wf2, bf2,
           reg_src, reg_dst, gp_src, gp_dst):
    rs = _pad_idx(reg_src, 2560)
    rd = _pad_idx(reg_dst, 2560)
    rw = _pad_w(reg_w, 2560)
    gs = _pad_idx(gp_src, 1280)
    gd = _pad_idx(gp_dst, 1280)
    gw = _pad_w(gp_w, 1280)

    aggp = _seg_sum_reg(gene_feat, rs, rd, rw)
    h_gene = _hgene(aggp, gene_feat, Wr, br.reshape(1, HID))
    accp = _seg_sum_gp(h_gene, gs, gd, gw)
    wsi = _wsi(patches, Wl, bl.reshape(1, HID), Wm, bm.reshape(1, HID),
               ln2_g.reshape(1, HID), ln2_b.reshape(1, HID)).reshape(NPAT, HID)
    logits, omics = _fusion(
        accp, wsi, Wp, bp.reshape(1, HID), ln1_g.reshape(1, HID),
        ln1_b.reshape(1, HID), Wo, bo.reshape(1, NBINS),
        Wf1[:HID], Wf1[HID:], bf1.reshape(1, HID), Wf2,
        bf2.reshape(1, NBINS))
    return (logits, omics)


# speedup vs baseline: 3.6661x; 1.3761x over previous
import functools

import jax
import jax.numpy as jnp
from jax import lax
from jax.experimental import pallas as pl
from jax.experimental.pallas import tpu as pltpu
from jax.experimental.pallas import tpu_sc as plsc

HID = 128
NBINS = 4
NGENE = 10000
NPAT = 64
EREG = 320000
EGP = 160000
NPATCH = 512
PDIM = 768

NC = 2
NS = 16
LANES = 16


def _gelu(x):
    return 0.5 * x * (1.0 + lax.erf(x * 0.7071067811865476))


def _make_seg_sum(n_idx_rows, nseg, half_rows):
    rows_per_tile = n_idx_rows // (NC * NS)
    assert rows_per_tile * NC * NS == n_idx_rows
    n_halves = rows_per_tile // half_rows
    assert n_halves * half_rows == rows_per_tile
    assert half_rows % 2 == 0
    zrows = nseg // NS
    assert zrows * NS == nseg
    UR = 128

    mesh = plsc.VectorSubcoreMesh(core_axis_name="c", subcore_axis_name="s",
                                  num_cores=NC, num_subcores=NS)

    @functools.partial(
        pl.kernel,
        out_type=jax.ShapeDtypeStruct((NC, nseg, HID), jnp.float32),
        mesh=mesh,
        scratch_types=[
            pltpu.VMEM((half_rows, 128), jnp.int32),
            pltpu.VMEM((half_rows, 128), jnp.int32),
            pltpu.VMEM((half_rows * 128,), jnp.float32),
            pltpu.VMEM((UR, HID), jnp.float32),
            pltpu.VMEM((UR, HID), jnp.float32),
            pltpu.VMEM_SHARED((nseg, HID), jnp.float32),
            pltpu.SemaphoreType.DMA,
            pltpu.SemaphoreType.DMA,
        ],
    )
    def seg_sum(feat_hbm, src_hbm, dst_hbm, w_hbm, out_hbm,
                src_v, dst_v, w_v, rows0, rows1, acc_sh, sem0, sem1):
        c = lax.axis_index("c")
        s = lax.axis_index("s")
        tile = c * NS + s
        row_base = tile * rows_per_tile
        bufs = (rows0, rows1)
        sems = (sem0, sem1)

        def zbody(r, carry):
            for j in range(HID // LANES):
                rows0[r, pl.ds(j * LANES, LANES)] = jnp.zeros((LANES,),
                                                              jnp.float32)
            return carry
        lax.fori_loop(0, UR, zbody, 0)
        off = 0
        rem = zrows
        while rem > 0:
            n = min(rem, UR)
            pltpu.sync_copy(rows0.at[pl.ds(0, n)],
                            acc_sh.at[pl.ds(s * zrows + off, n)])
            off += n
            rem -= n
        plsc.subcore_barrier()

        def scale(buf, u):
            def sbody(g, carry2):
                r0 = g * LANES
                w16 = w_v[pl.ds(u * 128 + r0, LANES)]
                for k in range(LANES):
                    wk = w16[k]
                    for j in range(HID // LANES):
                        sl = pl.ds(j * LANES, LANES)
                        buf[r0 + k, sl] = buf[r0 + k, sl] * wk
                return carry2
            lax.fori_loop(0, UR // LANES, sbody, 0)

        for h in range(n_halves):
            hb = row_base + h * half_rows
            pltpu.sync_copy(src_hbm.at[pl.ds(hb, half_rows)], src_v)
            pltpu.sync_copy(dst_hbm.at[pl.ds(hb, half_rows)], dst_v)
            pltpu.sync_copy(w_hbm.at[pl.ds(hb * 128, half_rows * 128)], w_v)
            pltpu.async_copy(feat_hbm.at[src_v.at[0]], rows0, sem0)

            def pair(i, carry):
                for b in range(2):
                    u = i * 2 + b
                    nb = (b + 1) % 2
                    @pl.when(u + 1 < half_rows)
                    def _():
                        pltpu.async_copy(feat_hbm.at[src_v.at[u + 1]],
                                         bufs[nb], sems[nb])
                    pltpu.make_async_copy(feat_hbm.at[src_v.at[u]],
                                          bufs[b], sems[b]).wait()
                    scale(bufs[b], u)
                    pltpu.sync_copy(bufs[b], acc_sh.at[dst_v.at[u]],
                                    add=True)
                return carry
            lax.fori_loop(0, half_rows // 2, pair, 0)

        plsc.subcore_barrier()
        off = 0
        rem = zrows
        while rem > 0:
            n = min(rem, UR)
            pltpu.sync_copy(acc_sh.at[pl.ds(s * zrows + off, n)],
                            out_hbm.at[c, pl.ds(s * zrows + off, n)])
            off += n
            rem -= n

    return seg_sum


_seg_sum_reg = _make_seg_sum(n_idx_rows=2560, nseg=10240, half_rows=40)
_seg_sum_gp = _make_seg_sum(n_idx_rows=1280, nseg=NPAT, half_rows=40)


def _hgene_body(agg_ref, gene_ref, wr_ref, br_ref, out_ref):
    a = agg_ref[0] + agg_ref[1]
    t = jnp.dot(a, wr_ref[...], preferred_element_type=jnp.float32)
    out_ref[...] = gene_ref[...] + _gelu(t + br_ref[...])


def _hgene(agg2, gene, Wr, br2):
    BR = 1000
    return pl.pallas_call(
        _hgene_body,
        grid=(NGENE // BR,),
        in_specs=[
            pl.BlockSpec((NC, BR, HID), lambda i: (0, i, 0)),
            pl.BlockSpec((BR, HID), lambda i: (i, 0)),
            pl.BlockSpec((HID, HID), lambda i: (0, 0)),
            pl.BlockSpec((1, HID), lambda i: (0, 0)),
        ],
        out_specs=pl.BlockSpec((BR, HID), lambda i: (i, 0)),
        out_shape=jax.ShapeDtypeStruct((NGENE, HID), jnp.float32),
    )(agg2, gene, Wr, br2)


def _wsi_body(p_ref, wl_ref, bl_ref, wm_ref, bm_ref, g_ref, b_ref, out_ref):
    x = p_ref[0]
    m = jnp.sum(x, axis=0, keepdims=True) * (1.0 / NPATCH)
    v = _gelu(jnp.dot(m, wl_ref[...], preferred_element_type=jnp.float32)
              + bl_ref[...])
    v = _gelu(jnp.dot(v, wm_ref[...], preferred_element_type=jnp.float32)
              + bm_ref[...])
    mu = jnp.mean(v, axis=-1, keepdims=True)
    var = jnp.mean((v - mu) ** 2, axis=-1, keepdims=True)
    out_ref[0] = (v - mu) / jnp.sqrt(var + 1e-5) * g_ref[...] + b_ref[...]


def _wsi(patches, Wl, bl2, Wm, bm2, g2, b2):
    return pl.pallas_call(
        _wsi_body,
        grid=(NPAT,),
        in_specs=[
            pl.BlockSpec((1, NPATCH, PDIM), lambda i: (i, 0, 0)),
            pl.BlockSpec((PDIM, HID), lambda i: (0, 0)),
            pl.BlockSpec((1, HID), lambda i: (0, 0)),
            pl.BlockSpec((HID, HID), lambda i: (0, 0)),
            pl.BlockSpec((1, HID), lambda i: (0, 0)),
            pl.BlockSpec((1, HID), lambda i: (0, 0)),
            pl.BlockSpec((1, HID), lambda i: (0, 0)),
        ],
        out_specs=pl.BlockSpec((1, 1, HID), lambda i: (i, 0, 0)),
        out_shape=jax.ShapeDtypeStruct((NPAT, 1, HID), jnp.float32),
    )(patches, Wl, bl2, Wm, bm2, g2, b2)


def _fusion_body(acc_ref, wsi_ref, wp_ref, bp_ref, g1_ref, b1_ref,
                 wo_ref, bo_ref, wf1a_ref, wf1b_ref, bf1_ref,
                 wf2_ref, bf2_ref, logits_ref, omics_ref):
    acc = acc_ref[0] + acc_ref[1]
    v = _gelu(jnp.dot(acc, wp_ref[...], preferred_element_type=jnp.float32)
              + bp_ref[...])
    mu = jnp.mean(v, axis=-1, keepdims=True)
    var = jnp.mean((v - mu) ** 2, axis=-1, keepdims=True)
    pat = (v - mu) / jnp.sqrt(var + 1e-5) * g1_ref[...] + b1_ref[...]
    omics_ref[...] = (jnp.dot(pat, wo_ref[...],
                              preferred_element_type=jnp.float32)
                      + bo_ref[...])
    h = _gelu(jnp.dot(pat, wf1a_ref[...], preferred_element_type=jnp.float32)
              + jnp.dot(wsi_ref[...], wf1b_ref[...],
                        preferred_element_type=jnp.float32)
              + bf1_ref[...])
    logits_ref[...] = (jnp.dot(h, wf2_ref[...],
                               preferred_element_type=jnp.float32)
                       + bf2_ref[...])


def _fusion(accp, wsi, Wp, bp2, g12, b12, Wo, bo2, Wf1a, Wf1b, bf12,
            Wf2, bf22):
    return pl.pallas_call(
        _fusion_body,
        out_shape=(jax.ShapeDtypeStruct((NPAT, NBINS), jnp.float32),
                   jax.ShapeDtypeStruct((NPAT, NBINS), jnp.float32)),
    )(accp, wsi, Wp, bp2, g12, b12, Wo, bo2, Wf1a, Wf1b, bf12, Wf2, bf22)


def _pad_idx(x, n_rows):
    pad = n_rows * 128 - x.shape[0]
    x = jnp.concatenate([x.astype(jnp.int32), jnp.zeros((pad,), jnp.int32)])
    return x.reshape(n_rows, 128)


def _pad_w(w, n_rows):
    pad = n_rows * 128 - w.shape[0]
    return jnp.concatenate([w, jnp.zeros((pad,), jnp.float32)])


def kernel(gene_feat, reg_w, gp_w, patches, Wr, br, Wp, bp, ln1_g, ln1_b,
           Wo, bo, Wl, bl, Wm, bm, ln2_g, ln2_b, Wf1, bf1, Wf2, bf2,
           reg_src, reg_dst, gp_src, gp_dst):
    rs = _pad_idx(reg_src, 2560)
    rd = _pad_idx(reg_dst, 2560)
    rw = _pad_w(reg_w, 2560)
    gs = _pad_idx(gp_src, 1280)
    gd = _pad_idx(gp_dst, 1280)
    gw = _pad_w(gp_w, 1280)

    aggp = _seg_sum_reg(gene_feat, rs, rd, rw)
    h_gene = _hgene(aggp, gene_feat, Wr, br.reshape(1, HID))
    accp = _seg_sum_gp(h_gene, gs, gd, gw)
    wsi = _wsi(patches, Wl, bl.reshape(1, HID), Wm, bm.reshape(1, HID),
               ln2_g.reshape(1, HID), ln2_b.reshape(1, HID)).reshape(NPAT, HID)
    logits, omics = _fusion(
        accp, wsi, Wp, bp.reshape(1, HID), ln1_g.reshape(1, HID),
        ln1_b.reshape(1, HID), Wo, bo.reshape(1, NBINS),
        Wf1[:HID], Wf1[HID:], bf1.reshape(1, HID), Wf2,
        bf2.reshape(1, NBINS))
    return (logits, omics)
